# baseline (speedup 1.0000x reference)
import functools

import jax
import jax.numpy as jnp
import numpy as np
from jax import lax
from jax.experimental import pallas as pl
from jax.experimental.pallas import tpu as pltpu
from jax.experimental.pallas import tpu_sc as plsc

_K = 16
_N = 100
_SIGMA = 0.05
_B = 8
_D = 2048
_L = 16
_NCH = _D // _L
_SPT = _N // 4


def _draw_noise():
    noise = jax.random.normal(jax.random.key(42), (_B, _N, _D), dtype=jnp.float32)
    return noise * jnp.float32(_SIGMA)


def _scaled_noise_host():
    try:
        with jax.default_device(jax.devices("cpu")[0]):
            return np.asarray(_draw_noise())
    except Exception:
        pass
    try:
        return np.asarray(_draw_noise())
    except Exception:
        return None


_SN_CONST = _scaled_noise_host()


def _sc_body(
    x_hbm, sn_hbm, out_hbm, xrow, slab, sidx, tidx, res, cvals, cidx, cvals2, cidx2, shared
):
    c = lax.axis_index("c")
    s = lax.axis_index("s")
    b = c * 4 + s // 4
    q = s % 4
    s0 = q * _SPT

    pltpu.sync_copy(x_hbm.at[b], xrow)
    pltpu.sync_copy(sn_hbm.at[b, pl.ds(s0, _SPT)], slab)

    lane = lax.iota(jnp.int32, _L)
    ones = jnp.full((_L,), 1.0, jnp.float32)
    neginf = jnp.full((_L,), -jnp.inf, jnp.float32)

    groups = [(1, 1), (2, 2), (4, 4), (8, 8)]
    groups += [(g, 16) for g in range(16, _NCH, 16)]

    fifteen = jnp.full((_L,), _L - 1, jnp.int32)

    def init_row(r):
        v0 = slab[r, pl.ds(0, _L)] + xrow[pl.ds(0, _L)]
        tv, ti = plsc.sort_key_val(v0, lane, descending=True)
        return tv, ti, jnp.take(tv, fifteen)

    def collect_n(r, t, cvr, cir, kks):
        cnt = jnp.int32(0)
        vs, ms, ss = [], [], []
        for kk in kks:
            base = kk * _L
            v = slab[r, pl.ds(base, _L)] + xrow[pl.ds(base, _L)]
            m = v > t
            vs.append((v, base))
            ms.append(m)
            ss.append(plsc.all_reduce_population_count(m)[0])
        offs = [cnt]
        for u in range(len(kks) - 1):
            offs.append(offs[-1] + ss[u])
        for u, (v, base) in enumerate(vs):
            plsc.store_compressed(cvr.at[pl.ds(offs[u], _L)], v, mask=ms[u])
            plsc.store_compressed(cir.at[pl.ds(offs[u], _L)], lane + base, mask=ms[u])
        return offs[-1] + ss[-1]

    def merge_group(tv, ti, t, cnt, cvr, cir):
        def bmerge(mi, carry):
            tv, ti = carry
            off = mi * _L
            cv = cvr[pl.ds(off, _L)]
            ci = cir[pl.ds(off, _L)]
            valid = (lane + off) < cnt
            cv = jnp.where(valid, cv, neginf)
            cv, ci = plsc.sort_key_val(cv, ci, descending=False)
            take = tv >= cv
            nv = jnp.where(take, tv, cv)
            ni = jnp.where(take, ti, ci)
            nv, ni = plsc.sort_key_val(nv, ni, descending=True)
            return nv, ni

        nb = (cnt + (_L - 1)) >> 4
        tv, ti = lax.fori_loop(0, nb, bmerge, (tv, ti))
        return tv, ti, jnp.take(tv, fifteen)

    def finish_row(r, tv, ti):
        si, _ = plsc.sort_key_val(ti, tv, descending=False)
        sidx[r] = si

    def pair_body(pi, _):
        ra = pi * 2
        rb = ra + 1
        tva, tia, ta = init_row(ra)
        tvb, tib, tb = init_row(rb)
        for g0, glen in groups:
            kks = list(range(g0, g0 + glen))
            cnta = collect_n(ra, ta, cvals, cidx, kks)
            cntb = collect_n(rb, tb, cvals2, cidx2, kks)
            tva, tia, ta = merge_group(tva, tia, ta, cnta, cvals, cidx)
            tvb, tib, tb = merge_group(tvb, tib, tb, cntb, cvals2, cidx2)
        finish_row(ra, tva, tia)
        finish_row(rb, tvb, tib)
        return 0

    lax.fori_loop(0, _SPT // 2, pair_body, 0)

    tva, tia, ta = init_row(_SPT - 1)
    for g0, glen in groups:
        cnta = collect_n(_SPT - 1, ta, cvals, cidx, list(range(g0, g0 + glen)))
        tva, tia, ta = merge_group(tva, tia, ta, cnta, cvals, cidx)
    finish_row(_SPT - 1, tva, tia)

    sp = (s // 4) * 4
    rowbase = q * 4
    pltpu.sync_copy(sidx, shared.at[s])
    plsc.subcore_barrier()

    zero = jnp.zeros((_L,), jnp.float32)

    def zbody(i, _):
        row = i // (_D // (8 * _L))
        c0 = (i % (_D // (8 * _L))) * (8 * _L)
        for u in range(8):
            res[row, pl.ds(c0 + u * _L, _L)] = zero
        return 0

    lax.fori_loop(0, 4 * _D // (8 * _L), zbody, 0)

    jvec = (lane - rowbase) & 3
    jmask = (lane >= rowbase) & (lane < rowbase + 4)
    ones = jnp.full((_L,), 1.0, jnp.float32)

    for pj in range(4):
        pltpu.sync_copy(shared.at[sp + pj], tidx)

        def scat(r, _):
            plsc.addupdate_scatter(res, [jvec, tidx[r]], ones, mask=jmask)
            return 0

        lax.fori_loop(0, _SPT, scat, 0)

    def scale(i, _):
        row = i // (_D // (8 * _L))
        c0 = (i % (_D // (8 * _L))) * (8 * _L)
        for u in range(8):
            col = c0 + u * _L
            res[row, pl.ds(col, _L)] = res[row, pl.ds(col, _L)] * jnp.float32(
                1.0 / _N
            )
        return 0

    lax.fori_loop(0, 4 * _D // (8 * _L), scale, 0)

    pltpu.sync_copy(res, out_hbm.at[b, pl.ds(rowbase, 4)])


@jax.jit
def _run(x, sn):
    mesh = plsc.VectorSubcoreMesh(core_axis_name="c", subcore_axis_name="s")
    return pl.kernel(
        _sc_body,
        out_type=jax.ShapeDtypeStruct((_B, _K, _D), jnp.float32),
        mesh=mesh,
        scratch_types=[
            pltpu.VMEM((_D,), jnp.float32),
            pltpu.VMEM((_SPT, _D), jnp.float32),
            pltpu.VMEM((_SPT, _L), jnp.int32),
            pltpu.VMEM((_SPT, _L), jnp.int32),
            pltpu.VMEM((4, _D), jnp.float32),
            pltpu.VMEM((272,), jnp.float32),
            pltpu.VMEM((272,), jnp.int32),
            pltpu.VMEM((272,), jnp.float32),
            pltpu.VMEM((272,), jnp.int32),
            pltpu.VMEM_SHARED((16, _SPT, _L), jnp.int32),
        ],
        compiler_params=pltpu.CompilerParams(
            use_tc_tiling_on_sc=False, needs_layout_passes=False
        ),
    )(x, sn)


def kernel(x):
    sn = jnp.asarray(_SN_CONST) if _SN_CONST is not None else _draw_noise()
    return _run(x, sn)


# speedup vs baseline: 1.5083x; 1.5083x over previous
import functools

import jax
import jax.numpy as jnp
import numpy as np
from jax import lax
from jax.experimental import pallas as pl
from jax.experimental.pallas import tpu as pltpu
from jax.experimental.pallas import tpu_sc as plsc

_K = 16
_N = 100
_SIGMA = 0.05
_B = 8
_D = 2048
_L = 16
_NCH = _D // _L
_SPT = _N // 4


def _draw_noise():
    noise = jax.random.normal(jax.random.key(42), (_B, _N, _D), dtype=jnp.float32)
    return noise * jnp.float32(_SIGMA)


def _scaled_noise_host():
    try:
        with jax.default_device(jax.devices("cpu")[0]):
            return np.asarray(_draw_noise())
    except Exception:
        pass
    try:
        return np.asarray(_draw_noise())
    except Exception:
        return None


_SN_CONST = _scaled_noise_host()


def _sc_body(x_hbm, sn_hbm, out_hbm, xrow, slab, sidx, tidx, res, cvals, cidx, shared):
    c = lax.axis_index("c")
    s = lax.axis_index("s")
    b = c * 4 + s // 4
    q = s % 4
    s0 = q * _SPT

    pltpu.sync_copy(x_hbm.at[b], xrow)
    pltpu.sync_copy(sn_hbm.at[b, pl.ds(s0, _SPT)], slab)

    lane = lax.iota(jnp.int32, _L)
    ones = jnp.full((_L,), 1.0, jnp.float32)
    neginf = jnp.full((_L,), -jnp.inf, jnp.float32)

    groups = [(1, 1), (2, 2), (4, 4), (8, 8)]
    groups += [(g, 16) for g in range(16, _NCH, 16)]

    fifteen = jnp.full((_L,), _L - 1, jnp.int32)

    def sample_body(r, _):
        v0 = slab[r, pl.ds(0, _L)] + xrow[pl.ds(0, _L)]
        tv, ti = plsc.sort_key_val(v0, lane, descending=True)
        t = jnp.take(tv, fifteen)

        def collect_n(cnt, kks):
            vs, ms, ss = [], [], []
            for kk in kks:
                base = kk * _L
                v = slab[r, pl.ds(base, _L)] + xrow[pl.ds(base, _L)]
                m = v > t
                vs.append((v, base))
                ms.append(m)
                ss.append(plsc.all_reduce_population_count(m)[0])
            offs = [cnt]
            for u in range(len(kks) - 1):
                offs.append(offs[-1] + ss[u])
            for u, (v, base) in enumerate(vs):
                plsc.store_compressed(cvals.at[pl.ds(offs[u], _L)], v, mask=ms[u])
                plsc.store_compressed(
                    cidx.at[pl.ds(offs[u], _L)], lane + base, mask=ms[u]
                )
            return offs[-1] + ss[-1]

        def bmerge(mi, carry):
            tv, ti, cnt = carry
            off = mi * _L
            cv = cvals[pl.ds(off, _L)]
            ci = cidx[pl.ds(off, _L)]
            valid = (lane + off) < cnt
            cv = jnp.where(valid, cv, neginf)
            cv, ci = plsc.sort_key_val(cv, ci, descending=False)
            take = tv >= cv
            nv = jnp.where(take, tv, cv)
            ni = jnp.where(take, ti, ci)
            nv, ni = plsc.sort_key_val(nv, ni, descending=True)
            return nv, ni, cnt

        for g0, glen in groups[:4]:
            cnt = collect_n(jnp.int32(0), list(range(g0, g0 + glen)))
            nb = (cnt + (_L - 1)) // _L
            tv, ti, _ = lax.fori_loop(0, nb, bmerge, (tv, ti, cnt))
            t = jnp.take(tv, fifteen)

        def main_group(g, carry):
            tv, ti, t = carry
            g0 = _L + g * _L

            def chunk16(t):
                ms, ss, bases = [], [], []
                for u in range(_L):
                    base = (g0 + u) * _L
                    v = slab[r, pl.ds(base, _L)] + xrow[pl.ds(base, _L)]
                    m = v > t
                    bases.append(base)
                    ms.append((v, m))
                    ss.append(plsc.all_reduce_population_count(m)[0])
                offs = [jnp.int32(0)]
                for u in range(_L - 1):
                    offs.append(offs[-1] + ss[u])
                for u, (v, m) in enumerate(ms):
                    plsc.store_compressed(cvals.at[pl.ds(offs[u], _L)], v, mask=m)
                    plsc.store_compressed(
                        cidx.at[pl.ds(offs[u], _L)], lane + bases[u], mask=m
                    )
                return offs[-1] + ss[-1]

            cnt = chunk16(t)
            nb = (cnt + (_L - 1)) >> 4
            tv, ti, _ = lax.fori_loop(0, nb, bmerge, (tv, ti, cnt))
            return tv, ti, jnp.take(tv, fifteen)

        tv, ti, t = lax.fori_loop(0, _NCH // _L - 1, main_group, (tv, ti, t))

        si, _ = plsc.sort_key_val(ti, tv, descending=False)
        sidx[r] = si
        return 0

    lax.fori_loop(0, _SPT, sample_body, 0)

    sp = (s // 4) * 4
    rowbase = q * 4
    pltpu.sync_copy(sidx, shared.at[s])
    plsc.subcore_barrier()

    zero = jnp.zeros((_L,), jnp.float32)

    def zbody(i, _):
        row = i // (_D // (8 * _L))
        c0 = (i % (_D // (8 * _L))) * (8 * _L)
        for u in range(8):
            res[row, pl.ds(c0 + u * _L, _L)] = zero
        return 0

    lax.fori_loop(0, 4 * _D // (8 * _L), zbody, 0)

    jvec = (lane - rowbase) & 3
    jmask = (lane >= rowbase) & (lane < rowbase + 4)
    ones = jnp.full((_L,), 1.0, jnp.float32)

    for pj in range(4):
        pltpu.sync_copy(shared.at[sp + pj], tidx)

        def scat(r, _):
            plsc.addupdate_scatter(res, [jvec, tidx[r]], ones, mask=jmask)
            return 0

        lax.fori_loop(0, _SPT, scat, 0)

    def scale(i, _):
        row = i // (_D // (8 * _L))
        c0 = (i % (_D // (8 * _L))) * (8 * _L)
        for u in range(8):
            col = c0 + u * _L
            res[row, pl.ds(col, _L)] = res[row, pl.ds(col, _L)] * jnp.float32(
                1.0 / _N
            )
        return 0

    lax.fori_loop(0, 4 * _D // (8 * _L), scale, 0)

    pltpu.sync_copy(res, out_hbm.at[b, pl.ds(rowbase, 4)])


@jax.jit
def _run(x, sn):
    mesh = plsc.VectorSubcoreMesh(core_axis_name="c", subcore_axis_name="s")
    return pl.kernel(
        _sc_body,
        out_type=jax.ShapeDtypeStruct((_B, _K, _D), jnp.float32),
        mesh=mesh,
        scratch_types=[
            pltpu.VMEM((_D,), jnp.float32),
            pltpu.VMEM((_SPT, _D), jnp.float32),
            pltpu.VMEM((_SPT, _L), jnp.int32),
            pltpu.VMEM((_SPT, _L), jnp.int32),
            pltpu.VMEM((4, _D), jnp.float32),
            pltpu.VMEM((272,), jnp.float32),
            pltpu.VMEM((272,), jnp.int32),
            pltpu.VMEM_SHARED((16, _SPT, _L), jnp.int32),
        ],
        compiler_params=pltpu.CompilerParams(
            use_tc_tiling_on_sc=False, needs_layout_passes=False
        ),
    )(x, sn)


def kernel(x):
    sn = jnp.asarray(_SN_CONST) if _SN_CONST is not None else _draw_noise()
    return _run(x, sn)


# speedup vs baseline: 1.5524x; 1.0292x over previous
import functools

import jax
import jax.numpy as jnp
import numpy as np
from jax import lax
from jax.experimental import pallas as pl
from jax.experimental.pallas import tpu as pltpu
from jax.experimental.pallas import tpu_sc as plsc

_K = 16
_N = 100
_SIGMA = 0.05
_B = 8
_D = 2048
_L = 16
_NCH = _D // _L
_SPT = _N // 4


def _draw_noise():
    noise = jax.random.normal(jax.random.key(42), (_B, _N, _D), dtype=jnp.float32)
    return noise * jnp.float32(_SIGMA)


def _scaled_noise_host():
    try:
        with jax.default_device(jax.devices("cpu")[0]):
            return np.asarray(_draw_noise())
    except Exception:
        pass
    try:
        return np.asarray(_draw_noise())
    except Exception:
        return None


_SN_CONST = _scaled_noise_host()


def _sc_body(x_hbm, sn_hbm, out_hbm, xrow, slab, sidx, tidx, res, cvals, cidx, shared):
    c = lax.axis_index("c")
    s = lax.axis_index("s")
    b = c * 4 + s // 4
    q = s % 4
    s0 = q * _SPT

    pltpu.sync_copy(x_hbm.at[b], xrow)
    pltpu.sync_copy(sn_hbm.at[b, pl.ds(s0, _SPT)], slab)

    lane = lax.iota(jnp.int32, _L)
    ones = jnp.full((_L,), 1.0, jnp.float32)
    neginf = jnp.full((_L,), -jnp.inf, jnp.float32)

    groups = [(1, 1), (2, 2), (4, 4), (8, 8)]
    groups += [(g, 16) for g in range(16, _NCH, 16)]

    fifteen = jnp.full((_L,), _L - 1, jnp.int32)

    def sample_body(r, _):
        v0 = slab[r, pl.ds(0, _L)] + xrow[pl.ds(0, _L)]
        tv, ti = plsc.sort_key_val(v0, lane, descending=True)
        t = jnp.take(tv, fifteen)

        def collect_n(cnt, kks):
            vs, ms, ss = [], [], []
            for kk in kks:
                base = kk * _L
                v = slab[r, pl.ds(base, _L)] + xrow[pl.ds(base, _L)]
                m = v > t
                vs.append((v, base))
                ms.append(m)
                ss.append(plsc.all_reduce_population_count(m)[0])
            offs = [cnt]
            for u in range(len(kks) - 1):
                offs.append(offs[-1] + ss[u])
            for u, (v, base) in enumerate(vs):
                plsc.store_compressed(cvals.at[pl.ds(offs[u], _L)], v, mask=ms[u])
                plsc.store_compressed(
                    cidx.at[pl.ds(offs[u], _L)], lane + base, mask=ms[u]
                )
            return offs[-1] + ss[-1]

        def bmerge(mi, carry):
            tv, ti, cnt = carry
            off = mi * _L
            cv = cvals[pl.ds(off, _L)]
            ci = cidx[pl.ds(off, _L)]
            valid = (lane + off) < cnt
            cv = jnp.where(valid, cv, neginf)
            cv, ci = plsc.sort_key_val(cv, ci, descending=False)
            take = tv >= cv
            nv = jnp.where(take, tv, cv)
            ni = jnp.where(take, ti, ci)
            nv, ni = plsc.sort_key_val(nv, ni, descending=True)
            return nv, ni, cnt

        for g0, glen in groups[:4]:
            cnt = collect_n(jnp.int32(0), list(range(g0, g0 + glen)))
            nb = (cnt + (_L - 1)) // _L
            tv, ti, _ = lax.fori_loop(0, nb, bmerge, (tv, ti, cnt))
            t = jnp.take(tv, fifteen)

        def main_tile(g, carry):
            tv, ti, t, cnt = carry
            g0 = _L + g * _L

            ms, ss, bases = [], [], []
            for u in range(_L):
                base = (g0 + u) * _L
                v = slab[r, pl.ds(base, _L)] + xrow[pl.ds(base, _L)]
                m = v > t
                bases.append(base)
                ms.append((v, m))
                ss.append(plsc.all_reduce_population_count(m)[0])
            offs = [cnt]
            for u in range(_L - 1):
                offs.append(offs[-1] + ss[u])
            for u, (v, m) in enumerate(ms):
                plsc.store_compressed(cvals.at[pl.ds(offs[u], _L)], v, mask=m)
                plsc.store_compressed(
                    cidx.at[pl.ds(offs[u], _L)], lane + bases[u], mask=m
                )
            cnt = offs[-1] + ss[-1]

            def with_merge(op):
                tv, ti, _, cnt = op
                nb = (cnt + (_L - 1)) >> 4
                tv, ti, _ = lax.fori_loop(0, nb, bmerge, (tv, ti, cnt))
                return tv, ti, jnp.take(tv, fifteen), jnp.int32(0)

            do_merge = (g == 0) | (g == 2) | (g == 6)
            return lax.cond(do_merge, with_merge, lambda op: op, (tv, ti, t, cnt))

        tv, ti, t, _ = lax.fori_loop(
            0, _NCH // _L - 1, main_tile, (tv, ti, t, jnp.int32(0))
        )

        si, _ = plsc.sort_key_val(ti, tv, descending=False)
        sidx[r] = si
        return 0

    lax.fori_loop(0, _SPT, sample_body, 0)

    sp = (s // 4) * 4
    rowbase = q * 4
    pltpu.sync_copy(sidx, shared.at[s])
    plsc.subcore_barrier()

    zero = jnp.zeros((_L,), jnp.float32)

    def zbody(i, _):
        row = i // (_D // (8 * _L))
        c0 = (i % (_D // (8 * _L))) * (8 * _L)
        for u in range(8):
            res[row, pl.ds(c0 + u * _L, _L)] = zero
        return 0

    lax.fori_loop(0, 4 * _D // (8 * _L), zbody, 0)

    jvec = (lane - rowbase) & 3
    jmask = (lane >= rowbase) & (lane < rowbase + 4)
    ones = jnp.full((_L,), 1.0, jnp.float32)

    for pj in range(4):
        pltpu.sync_copy(shared.at[sp + pj], tidx)

        def scat(r, _):
            plsc.addupdate_scatter(res, [jvec, tidx[r]], ones, mask=jmask)
            return 0

        lax.fori_loop(0, _SPT, scat, 0)

    def scale(i, _):
        row = i // (_D // (8 * _L))
        c0 = (i % (_D // (8 * _L))) * (8 * _L)
        for u in range(8):
            col = c0 + u * _L
            res[row, pl.ds(col, _L)] = res[row, pl.ds(col, _L)] * jnp.float32(
                1.0 / _N
            )
        return 0

    lax.fori_loop(0, 4 * _D // (8 * _L), scale, 0)

    pltpu.sync_copy(res, out_hbm.at[b, pl.ds(rowbase, 4)])


@jax.jit
def _run(x, sn):
    mesh = plsc.VectorSubcoreMesh(core_axis_name="c", subcore_axis_name="s")
    return pl.kernel(
        _sc_body,
        out_type=jax.ShapeDtypeStruct((_B, _K, _D), jnp.float32),
        mesh=mesh,
        scratch_types=[
            pltpu.VMEM((_D,), jnp.float32),
            pltpu.VMEM((_SPT, _D), jnp.float32),
            pltpu.VMEM((_SPT, _L), jnp.int32),
            pltpu.VMEM((_SPT, _L), jnp.int32),
            pltpu.VMEM((4, _D), jnp.float32),
            pltpu.VMEM((1040,), jnp.float32),
            pltpu.VMEM((1040,), jnp.int32),
            pltpu.VMEM_SHARED((16, _SPT, _L), jnp.int32),
        ],
        compiler_params=pltpu.CompilerParams(
            use_tc_tiling_on_sc=False, needs_layout_passes=False
        ),
    )(x, sn)


def kernel(x):
    sn = jnp.asarray(_SN_CONST) if _SN_CONST is not None else _draw_noise()
    return _run(x, sn)


# speedup vs baseline: 1.5624x; 1.0064x over previous
import functools

import jax
import jax.numpy as jnp
import numpy as np
from jax import lax
from jax.experimental import pallas as pl
from jax.experimental.pallas import tpu as pltpu
from jax.experimental.pallas import tpu_sc as plsc

_K = 16
_N = 100
_SIGMA = 0.05
_B = 8
_D = 2048
_L = 16
_NCH = _D // _L
_SPT = _N // 4


def _draw_noise():
    noise = jax.random.normal(jax.random.key(42), (_B, _N, _D), dtype=jnp.float32)
    return noise * jnp.float32(_SIGMA)


def _scaled_noise_host():
    try:
        with jax.default_device(jax.devices("cpu")[0]):
            return np.asarray(_draw_noise())
    except Exception:
        pass
    try:
        return np.asarray(_draw_noise())
    except Exception:
        return None


_SN_CONST = _scaled_noise_host()


def _sc_body(x_hbm, sn_hbm, out_hbm, xrow, slab, sidx, tidx, res, cvals, cidx, shared):
    c = lax.axis_index("c")
    s = lax.axis_index("s")
    b = c * 4 + s // 4
    q = s % 4
    s0 = q * _SPT

    pltpu.sync_copy(x_hbm.at[b], xrow)
    pltpu.sync_copy(sn_hbm.at[b, pl.ds(s0, _SPT)], slab)

    lane = lax.iota(jnp.int32, _L)
    ones = jnp.full((_L,), 1.0, jnp.float32)
    neginf = jnp.full((_L,), -jnp.inf, jnp.float32)

    groups = [(1, 1), (2, 2), (4, 4), (8, 8)]
    groups += [(g, 16) for g in range(16, _NCH, 16)]

    fifteen = jnp.full((_L,), _L - 1, jnp.int32)

    def sample_body(r, _):
        v0 = slab[r, pl.ds(0, _L)] + xrow[pl.ds(0, _L)]
        tv, ti = plsc.sort_key_val(v0, lane, descending=True)
        t = jnp.take(tv, fifteen)

        def collect_n(cnt, kks):
            vs, ms, ss = [], [], []
            for kk in kks:
                base = kk * _L
                v = slab[r, pl.ds(base, _L)] + xrow[pl.ds(base, _L)]
                m = v > t
                vs.append((v, base))
                ms.append(m)
                ss.append(plsc.all_reduce_population_count(m)[0])
            offs = [cnt]
            for u in range(len(kks) - 1):
                offs.append(offs[-1] + ss[u])
            for u, (v, base) in enumerate(vs):
                plsc.store_compressed(cvals.at[pl.ds(offs[u], _L)], v, mask=ms[u])
                plsc.store_compressed(
                    cidx.at[pl.ds(offs[u], _L)], lane + base, mask=ms[u]
                )
            return offs[-1] + ss[-1]

        def bmerge(mi, carry):
            tv, ti, cnt = carry
            off = mi * _L
            cv = cvals[pl.ds(off, _L)]
            ci = cidx[pl.ds(off, _L)]
            valid = (lane + off) < cnt
            cv = jnp.where(valid, cv, neginf)
            cv, ci = plsc.sort_key_val(cv, ci, descending=False)
            take = tv >= cv
            nv = jnp.where(take, tv, cv)
            ni = jnp.where(take, ti, ci)
            nv, ni = plsc.sort_key_val(nv, ni, descending=True)
            return nv, ni, cnt

        for g0, glen in groups[:4]:
            cnt = collect_n(jnp.int32(0), list(range(g0, g0 + glen)))
            nb = (cnt + (_L - 1)) // _L
            tv, ti, _ = lax.fori_loop(0, nb, bmerge, (tv, ti, cnt))
            t = jnp.take(tv, fifteen)

        def main_tile(g, carry):
            tv, ti, t, cnt = carry
            g0 = _L + g * _L

            ms, ss, bases = [], [], []
            for u in range(_L):
                base = (g0 + u) * _L
                v = slab[r, pl.ds(base, _L)] + xrow[pl.ds(base, _L)]
                m = v > t
                bases.append(base)
                ms.append((v, m))
                ss.append(plsc.all_reduce_population_count(m)[0])
            offs = [cnt]
            for u in range(_L - 1):
                offs.append(offs[-1] + ss[u])
            for u, (v, m) in enumerate(ms):
                plsc.store_compressed(cvals.at[pl.ds(offs[u], _L)], v, mask=m)
                plsc.store_compressed(
                    cidx.at[pl.ds(offs[u], _L)], lane + bases[u], mask=m
                )
            cnt = offs[-1] + ss[-1]

            def with_merge(op):
                tv, ti, _, cnt = op
                nb = (cnt + (_L - 1)) >> 4
                tv, ti, _ = lax.fori_loop(0, nb, bmerge, (tv, ti, cnt))
                return tv, ti, jnp.take(tv, fifteen), jnp.int32(0)

            do_merge = (g == 0) | (g == 2) | (g == 6)
            return lax.cond(do_merge, with_merge, lambda op: op, (tv, ti, t, cnt))

        tv, ti, t, _ = lax.fori_loop(
            0, _NCH // _L - 1, main_tile, (tv, ti, t, jnp.int32(0))
        )

        si, _ = plsc.sort_key_val(ti, tv, descending=False)
        sidx[r] = si
        return 0

    lax.fori_loop(0, _SPT, sample_body, 0)

    sp = (s // 4) * 4
    rowbase = q * 4
    pltpu.sync_copy(sidx, shared.at[s])

    zero = jnp.zeros((_L,), jnp.float32)

    def zbody(i, _):
        row = i // (_D // (8 * _L))
        c0 = (i % (_D // (8 * _L))) * (8 * _L)
        for u in range(8):
            res[row, pl.ds(c0 + u * _L, _L)] = zero
        return 0

    lax.fori_loop(0, 4 * _D // (8 * _L), zbody, 0)

    plsc.subcore_barrier()

    jvec = (lane - rowbase) & 3
    jmask = (lane >= rowbase) & (lane < rowbase + 4)
    ones = jnp.full((_L,), 1.0, jnp.float32)

    pltpu.sync_copy(shared.at[pl.ds(sp, 4)], tidx)
    for pj in range(4):

        def scat(r, _, pj=pj):
            plsc.addupdate_scatter(res, [jvec, tidx[pj, r]], ones, mask=jmask)
            return 0

        lax.fori_loop(0, _SPT, scat, 0)

    def scale(i, _):
        row = i // (_D // (8 * _L))
        c0 = (i % (_D // (8 * _L))) * (8 * _L)
        for u in range(8):
            col = c0 + u * _L
            res[row, pl.ds(col, _L)] = res[row, pl.ds(col, _L)] * jnp.float32(
                1.0 / _N
            )
        return 0

    lax.fori_loop(0, 4 * _D // (8 * _L), scale, 0)

    pltpu.sync_copy(res, out_hbm.at[b, pl.ds(rowbase, 4)])


@jax.jit
def _run(x, sn):
    mesh = plsc.VectorSubcoreMesh(core_axis_name="c", subcore_axis_name="s")
    return pl.kernel(
        _sc_body,
        out_type=jax.ShapeDtypeStruct((_B, _K, _D), jnp.float32),
        mesh=mesh,
        scratch_types=[
            pltpu.VMEM((_D,), jnp.float32),
            pltpu.VMEM((_SPT, _D), jnp.float32),
            pltpu.VMEM((_SPT, _L), jnp.int32),
            pltpu.VMEM((4, _SPT, _L), jnp.int32),
            pltpu.VMEM((4, _D), jnp.float32),
            pltpu.VMEM((1040,), jnp.float32),
            pltpu.VMEM((1040,), jnp.int32),
            pltpu.VMEM_SHARED((16, _SPT, _L), jnp.int32),
        ],
        compiler_params=pltpu.CompilerParams(
            use_tc_tiling_on_sc=False, needs_layout_passes=False
        ),
    )(x, sn)


def kernel(x):
    sn = jnp.asarray(_SN_CONST) if _SN_CONST is not None else _draw_noise()
    return _run(x, sn)


# speedup vs baseline: 1.8983x; 1.2150x over previous
import functools

import jax
import jax.numpy as jnp
import numpy as np
from jax import lax
from jax.experimental import pallas as pl
from jax.experimental.pallas import tpu as pltpu
from jax.experimental.pallas import tpu_sc as plsc

_K = 16
_N = 100
_SIGMA = 0.05
_B = 8
_D = 2048
_L = 16
_NCH = _D // _L
_SPT = _N // 4


def _draw_noise():
    noise = jax.random.normal(jax.random.key(42), (_B, _N, _D), dtype=jnp.float32)
    return noise * jnp.float32(_SIGMA)


def _scaled_noise_host():
    try:
        with jax.default_device(jax.devices("cpu")[0]):
            return np.asarray(_draw_noise())
    except Exception:
        pass
    try:
        return np.asarray(_draw_noise())
    except Exception:
        return None


_SN_CONST = _scaled_noise_host()


def _sc_body(
    x_hbm, sn_hbm, tm_hbm, out_hbm,
    xrow, slab, sidx, tidx, res, cvals, cidx, slist, tmv, shared,
):
    c = lax.axis_index("c")
    s = lax.axis_index("s")
    b = c * 4 + s // 4
    q = s % 4
    s0 = q * _SPT

    pltpu.sync_copy(x_hbm.at[b], xrow)
    pltpu.sync_copy(sn_hbm.at[b, pl.ds(s0, _SPT)], slab)
    pltpu.sync_copy(tm_hbm, tmv)

    lane = lax.iota(jnp.int32, _L)
    ones = jnp.full((_L,), 1.0, jnp.float32)
    neginf = jnp.full((_L,), -jnp.inf, jnp.float32)

    groups = [(1, 1), (2, 2), (4, 4), (8, 8)]
    groups += [(g, 16) for g in range(16, _NCH, 16)]

    fifteen = jnp.full((_L,), _L - 1, jnp.int32)

    def x_topk():
        v0 = xrow[pl.ds(0, _L)]
        tv, ti = plsc.sort_key_val(v0, lane, descending=True)
        t = jnp.take(tv, fifteen)

        def collect_n(cnt, kks):
            vs, ms, ss = [], [], []
            for kk in kks:
                base = kk * _L
                v = xrow[pl.ds(base, _L)]
                m = v > t
                vs.append((v, base))
                ms.append(m)
                ss.append(plsc.all_reduce_population_count(m)[0])
            offs = [cnt]
            for u in range(len(kks) - 1):
                offs.append(offs[-1] + ss[u])
            for u, (v, base) in enumerate(vs):
                plsc.store_compressed(cvals.at[pl.ds(offs[u], _L)], v, mask=ms[u])
                plsc.store_compressed(
                    cidx.at[pl.ds(offs[u], _L)], lane + base, mask=ms[u]
                )
            return offs[-1] + ss[-1]

        def bmerge(mi, carry):
            tv, ti, cnt = carry
            off = mi * _L
            cv = cvals[pl.ds(off, _L)]
            ci = cidx[pl.ds(off, _L)]
            valid = (lane + off) < cnt
            cv = jnp.where(valid, cv, neginf)
            cv, ci = plsc.sort_key_val(cv, ci, descending=False)
            take = tv >= cv
            nv = jnp.where(take, tv, cv)
            ni = jnp.where(take, ti, ci)
            nv, ni = plsc.sort_key_val(nv, ni, descending=True)
            return nv, ni, cnt

        for g0, glen in groups[:4]:
            cnt = collect_n(jnp.int32(0), list(range(g0, g0 + glen)))
            nb = (cnt + (_L - 1)) // _L
            tv, ti, _ = lax.fori_loop(0, nb, bmerge, (tv, ti, cnt))
            t = jnp.take(tv, fifteen)

        def main_tile(g, carry):
            tv, ti, t, cnt = carry
            g0 = _L + g * _L

            ms, ss, bases = [], [], []
            for u in range(_L):
                base = (g0 + u) * _L
                v = xrow[pl.ds(base, _L)]
                m = v > t
                bases.append(base)
                ms.append((v, m))
                ss.append(plsc.all_reduce_population_count(m)[0])
            offs = [cnt]
            for u in range(_L - 1):
                offs.append(offs[-1] + ss[u])
            for u, (v, m) in enumerate(ms):
                plsc.store_compressed(cvals.at[pl.ds(offs[u], _L)], v, mask=m)
                plsc.store_compressed(
                    cidx.at[pl.ds(offs[u], _L)], lane + bases[u], mask=m
                )
            cnt = offs[-1] + ss[-1]

            def with_merge(op):
                tv, ti, _, cnt = op
                nb = (cnt + (_L - 1)) >> 4
                tv, ti, _ = lax.fori_loop(0, nb, bmerge, (tv, ti, cnt))
                return tv, ti, jnp.take(tv, fifteen), jnp.int32(0)

            do_merge = (g == 0) | (g == 2) | (g == 6)
            return lax.cond(do_merge, with_merge, lambda op: op, (tv, ti, t, cnt))

        tv, ti, t, _ = lax.fori_loop(
            0, _NCH // _L - 1, main_tile, (tv, ti, t, jnp.int32(0))
        )
        return t

    t16 = x_topk()
    tau = t16 - tmv[pl.ds(0, _L)]

    def sbody(kk, cnt):
        base = kk * _L
        v = xrow[pl.ds(base, _L)]
        m = v >= tau
        plsc.store_compressed(slist.at[pl.ds(cnt, _L)], lane + base, mask=m)
        return cnt + plsc.all_reduce_population_count(m)[0]

    ns = lax.fori_loop(0, _NCH, sbody, jnp.int32(0))
    slist[pl.ds(ns, _L)] = jnp.zeros((_L,), jnp.int32)
    nch_c = (ns + _L - 1) >> 4

    def comp_row(r, _):
        rf = lane * 0 + r

        def cc(i, _):
            idxv = slist[pl.ds(i * _L, _L)]
            gv = plsc.load_gather(slab, [rf, idxv]) + plsc.load_gather(
                xrow, [idxv]
            )
            slab[r, pl.ds(i * _L, _L)] = gv
            return 0

        lax.fori_loop(0, nch_c, cc, 0)

        @pl.when((ns & (_L - 1)) != 0)
        def _():
            pb = ns & ~(_L - 1)
            vv = slab[r, pl.ds(pb, _L)]
            mm = (pb + lane) >= ns
            slab[r, pl.ds(pb, _L)] = jnp.where(mm, neginf, vv)

        return 0

    lax.fori_loop(0, _SPT, comp_row, 0)

    def sample_body(r, _):
        cv0 = slab[r, pl.ds(0, _L)]
        tv, ti = plsc.sort_key_val(cv0, lane, descending=True)
        t = jnp.take(tv, fifteen)

        def cb(kk, carry):
            tv, ti, t = carry
            base = kk * _L
            v = slab[r, pl.ds(base, _L)]
            hit = jnp.any(v > t)

            def mg(op):
                tv, ti, _ = op
                cvv, cii = plsc.sort_key_val(v, lane + base, descending=False)
                tk = tv >= cvv
                nv = jnp.where(tk, tv, cvv)
                ni = jnp.where(tk, ti, cii)
                nv, ni = plsc.sort_key_val(nv, ni, descending=True)
                return nv, ni, jnp.take(nv, fifteen)

            return lax.cond(hit, mg, lambda op: op, (tv, ti, t))

        tv, ti, t = lax.fori_loop(1, nch_c, cb, (tv, ti, t))
        sic, _ = plsc.sort_key_val(ti, tv, descending=False)
        sidx[r] = plsc.load_gather(slist, [sic])
        return 0

    lax.fori_loop(0, _SPT, sample_body, 0)

    sp = (s // 4) * 4
    rowbase = q * 4
    pltpu.sync_copy(sidx, shared.at[s])

    zero = jnp.zeros((_L,), jnp.float32)

    def zbody(i, _):
        row = i // (_D // (8 * _L))
        c0 = (i % (_D // (8 * _L))) * (8 * _L)
        for u in range(8):
            res[row, pl.ds(c0 + u * _L, _L)] = zero
        return 0

    lax.fori_loop(0, 4 * _D // (8 * _L), zbody, 0)

    plsc.subcore_barrier()

    jvec = (lane - rowbase) & 3
    jmask = (lane >= rowbase) & (lane < rowbase + 4)
    ones = jnp.full((_L,), 1.0, jnp.float32)

    pltpu.sync_copy(shared.at[pl.ds(sp, 4)], tidx)
    for pj in range(4):

        def scat(r, _, pj=pj):
            plsc.addupdate_scatter(res, [jvec, tidx[pj, r]], ones, mask=jmask)
            return 0

        lax.fori_loop(0, _SPT, scat, 0)

    def scale(i, _):
        row = i // (_D // (8 * _L))
        c0 = (i % (_D // (8 * _L))) * (8 * _L)
        for u in range(8):
            col = c0 + u * _L
            res[row, pl.ds(col, _L)] = res[row, pl.ds(col, _L)] * jnp.float32(
                1.0 / _N
            )
        return 0

    lax.fori_loop(0, 4 * _D // (8 * _L), scale, 0)

    pltpu.sync_copy(res, out_hbm.at[b, pl.ds(rowbase, 4)])


@jax.jit
def _run(x, sn, tm):
    mesh = plsc.VectorSubcoreMesh(core_axis_name="c", subcore_axis_name="s")
    return pl.kernel(
        _sc_body,
        out_type=jax.ShapeDtypeStruct((_B, _K, _D), jnp.float32),
        mesh=mesh,
        scratch_types=[
            pltpu.VMEM((_D,), jnp.float32),
            pltpu.VMEM((_SPT, _D), jnp.float32),
            pltpu.VMEM((_SPT, _L), jnp.int32),
            pltpu.VMEM((4, _SPT, _L), jnp.int32),
            pltpu.VMEM((4, _D), jnp.float32),
            pltpu.VMEM((1040,), jnp.float32),
            pltpu.VMEM((1040,), jnp.int32),
            pltpu.VMEM((_D + _L,), jnp.int32),
            pltpu.VMEM((_L,), jnp.float32),
            pltpu.VMEM_SHARED((16, _SPT, _L), jnp.int32),
        ],
        compiler_params=pltpu.CompilerParams(
            use_tc_tiling_on_sc=False, needs_layout_passes=False
        ),
    )(x, sn, tm)


if _SN_CONST is not None:
    _TM_CONST = np.full(
        (_L,), 2.0 * float(np.max(np.abs(_SN_CONST))) + 1e-3, np.float32
    )
else:
    _TM_CONST = None


def kernel(x):
    if _SN_CONST is not None:
        sn = jnp.asarray(_SN_CONST)
        tm = jnp.asarray(_TM_CONST)
    else:
        sn = _draw_noise()
        tm = jnp.full((_L,), 2.0 * jnp.max(jnp.abs(sn)) + 1e-3, jnp.float32)
    return _run(x, sn, tm)
